# Initial kernel scaffold; baseline (speedup 1.0000x reference)
import jax
import jax.numpy as jnp
from jax.experimental import pallas as pl


def kernel(x, edge_index, edge_weight, W1, b1, W2, b2):
    raise NotImplementedError("write your pallas kernel here")


# speedup vs baseline: 41.3631x; 41.3631x over previous
import functools

import jax
import jax.numpy as jnp
from jax import lax
from jax.experimental import pallas as pl
from jax.experimental.pallas import tpu as pltpu
from jax.experimental.pallas import tpu_sc as plsc

N = 10000
E = 320000
D = 16
NC = 2
NS = 16
NW = NC * NS
SUB = 125
CH = 2000
NSUB = CH // SUB
EPW = E // NW
NCH = EPW // CH
RPT = 640
NP = NS * RPT

_mesh = plsc.VectorSubcoreMesh(
    core_axis_name="c", subcore_axis_name="s", num_cores=NC, num_subcores=NS)


@functools.partial(
    pl.kernel,
    out_type=jax.ShapeDtypeStruct((NC, NP), jnp.float32),
    mesh=_mesh,
    scratch_types=[
        pltpu.VMEM((NSUB, SUB), jnp.int32),
        pltpu.VMEM((NSUB, SUB), jnp.float32),
        pltpu.VMEM((RPT,), jnp.float32),
        pltpu.VMEM_SHARED((NP,), jnp.float32),
    ],
)
def _deg_kernel(col2, ew2, out, cidx_v, ew_v, zbuf, deg_s):
    c = lax.axis_index("c")
    s = lax.axis_index("s")
    wid = s * NC + c

    def _z(i, carry):
        zbuf[pl.ds(i * 16, 16)] = jnp.zeros((16,), jnp.float32)
        return carry

    lax.fori_loop(0, RPT // 16, _z, 0)
    pltpu.sync_copy(zbuf, deg_s.at[pl.ds(s * RPT, RPT)])
    plsc.subcore_barrier()

    def _chunk(k, carry):
        rb = wid * (EPW // SUB) + k * NSUB
        pltpu.sync_copy(col2.at[pl.ds(rb, NSUB)], cidx_v)
        pltpu.sync_copy(ew2.at[pl.ds(rb, NSUB)], ew_v)
        for j in range(NSUB):
            pltpu.sync_copy(ew_v.at[j], deg_s.at[cidx_v.at[j]], add=True)
        return carry

    lax.fori_loop(0, NCH, _chunk, 0)
    plsc.subcore_barrier()
    pltpu.sync_copy(deg_s.at[pl.ds(s * RPT, RPT)],
                    out.at[c, pl.ds(s * RPT, RPT)])


@functools.partial(
    pl.kernel,
    out_type=jax.ShapeDtypeStruct((NC, NP, D), jnp.float32),
    mesh=_mesh,
    scratch_types=[
        pltpu.VMEM((NSUB, SUB), jnp.int32),
        pltpu.VMEM((NSUB, SUB), jnp.int32),
        pltpu.VMEM((CH,), jnp.float32),
        pltpu.VMEM((CH, D), jnp.float32),
        pltpu.VMEM((RPT, D), jnp.float32),
        pltpu.VMEM_SHARED((NP, D), jnp.float32),
        pltpu.SemaphoreType.DMA,
    ],
    compiler_params=pltpu.CompilerParams(use_tc_tiling_on_sc=False),
)
def _prop_kernel(h, row2, col2, ew1, out, ridx_v, cidx_v, ew_v, rows_v,
                 zbuf, acc_s, sem):
    c = lax.axis_index("c")
    s = lax.axis_index("s")
    wid = s * NC + c

    def _z(i, carry):
        zbuf[i, :] = jnp.zeros((D,), jnp.float32)
        return carry

    lax.fori_loop(0, RPT, _z, 0)
    pltpu.sync_copy(zbuf, acc_s.at[pl.ds(s * RPT, RPT)])
    plsc.subcore_barrier()

    def _chunk(k, carry):
        rb = wid * (EPW // SUB) + k * NSUB
        pltpu.sync_copy(row2.at[pl.ds(rb, NSUB)], ridx_v)
        pltpu.sync_copy(col2.at[pl.ds(rb, NSUB)], cidx_v)
        pltpu.sync_copy(ew1.at[pl.ds(wid * EPW + k * CH, CH)], ew_v)
        descs = [
            pltpu.async_copy(h.at[ridx_v.at[j]],
                             rows_v.at[pl.ds(j * SUB, SUB)], sem)
            for j in range(NSUB)
        ]
        for d_ in descs:
            d_.wait()

        def _m(g, carry2):
            ew16 = ew_v[pl.ds(g * 16, 16)]
            base = g * 16
            for e in range(16):
                rows_v[base + e, :] = rows_v[base + e, :] * ew16[e]
            return carry2

        lax.fori_loop(0, CH // 16, _m, 0)
        for j in range(NSUB):
            pltpu.sync_copy(rows_v.at[pl.ds(j * SUB, SUB)],
                            acc_s.at[cidx_v.at[j]], add=True)
        return carry

    lax.fori_loop(0, NCH, _chunk, 0)
    plsc.subcore_barrier()
    pltpu.sync_copy(acc_s.at[pl.ds(s * RPT, RPT)],
                    out.at[c, pl.ds(s * RPT, RPT)])


def _tc1_body(x_ref, w1_ref, dega_ref, degb_ref, h1p_ref, dis16_ref):
    deg = dega_ref[...] + degb_ref[...] + 1.0
    dis = lax.rsqrt(deg)
    h = jnp.dot(x_ref[...], w1_ref[...], preferred_element_type=jnp.float32)
    h1p_ref[...] = dis * h
    dis16_ref[...] = jnp.broadcast_to(dis, (N, D))


_tc1 = pl.pallas_call(
    _tc1_body,
    out_shape=(jax.ShapeDtypeStruct((N, D), jnp.float32),
               jax.ShapeDtypeStruct((N, D), jnp.float32)),
)


def _tc2_body(p1a_ref, p1b_ref, h1p_ref, dis16_ref, w2_ref, b1_ref, h2p_ref):
    s = p1a_ref[...] + p1b_ref[...] + h1p_ref[...]
    out1 = dis16_ref[...] * s + b1_ref[...]
    a = jnp.maximum(out1, 0.0)
    h2p_ref[...] = dis16_ref[...] * jnp.dot(
        a, w2_ref[...], preferred_element_type=jnp.float32)


_tc2 = pl.pallas_call(
    _tc2_body,
    out_shape=jax.ShapeDtypeStruct((N, D), jnp.float32),
)


def _tc3_body(p2a_ref, p2b_ref, h2p_ref, dis16_ref, b2_ref, out_ref):
    z = dis16_ref[...] * (p2a_ref[...] + p2b_ref[...] + h2p_ref[...])
    z = z + b2_ref[...]
    m = jnp.max(z, axis=1, keepdims=True)
    lse = jnp.log(jnp.sum(jnp.exp(z - m), axis=1, keepdims=True)) + m
    out_ref[...] = z - lse


_tc3 = pl.pallas_call(
    _tc3_body,
    out_shape=jax.ShapeDtypeStruct((N, D), jnp.float32),
)


def kernel(x, edge_index, edge_weight, W1, b1, W2, b2):
    row2 = edge_index[0].astype(jnp.int32).reshape(E // SUB, SUB)
    col2 = edge_index[1].astype(jnp.int32).reshape(E // SUB, SUB)
    ew2 = edge_weight.reshape(E // SUB, SUB)

    degp = _deg_kernel(col2, ew2)
    dega = degp[0, :N].reshape(N, 1)
    degb = degp[1, :N].reshape(N, 1)
    h1p, dis16 = _tc1(x, W1, dega, degb)

    ew1 = edge_weight.reshape(E)
    p1 = _prop_kernel(h1p, row2, col2, ew1)
    h2p = _tc2(p1[0, :N], p1[1, :N], h1p, dis16, W2, b1.reshape(1, D))

    p2 = _prop_kernel(h2p, row2, col2, ew1)
    out = _tc3(p2[0, :N], p2[1, :N], h2p, dis16, b2.reshape(1, D))
    return out


# speedup vs baseline: 53.6528x; 1.2971x over previous
import functools

import jax
import jax.numpy as jnp
from jax import lax
from jax.experimental import pallas as pl
from jax.experimental.pallas import tpu as pltpu
from jax.experimental.pallas import tpu_sc as plsc

N = 10000
E = 320000
D = 16
NC = 2
NS = 16
NW = NC * NS
SUB = 125
CH = 2000
NSUB = CH // SUB
EPW = E // NW
NCH = EPW // CH
RPT = 640
NP = NS * RPT

_mesh = plsc.VectorSubcoreMesh(
    core_axis_name="c", subcore_axis_name="s", num_cores=NC, num_subcores=NS)


_RPS = EPW // SUB


@functools.partial(
    pl.kernel,
    out_type=jax.ShapeDtypeStruct((NC, NP), jnp.float32),
    mesh=_mesh,
    scratch_types=[
        pltpu.VMEM((_RPS, SUB), jnp.int32),
        pltpu.VMEM((_RPS, SUB), jnp.float32),
        pltpu.VMEM((RPT,), jnp.float32),
        pltpu.VMEM_SHARED((NP,), jnp.float32),
        pltpu.SemaphoreType.DMA,
    ],
)
def _deg_kernel(col2, ew2, out, cidx_v, ew_v, zbuf, deg_s, sem):
    c = lax.axis_index("c")
    s = lax.axis_index("s")
    wid = s * NC + c

    def _z(i, carry):
        zbuf[pl.ds(i * 16, 16)] = jnp.zeros((16,), jnp.float32)
        return carry

    lax.fori_loop(0, RPT // 16, _z, 0)
    pltpu.sync_copy(zbuf, deg_s.at[pl.ds(s * RPT, RPT)])
    rb = wid * _RPS
    pltpu.sync_copy(col2.at[pl.ds(rb, _RPS)], cidx_v)
    pltpu.sync_copy(ew2.at[pl.ds(rb, _RPS)], ew_v)
    plsc.subcore_barrier()
    descs = [pltpu.async_copy(ew_v.at[j], deg_s.at[cidx_v.at[j]], sem,
                              add=True)
             for j in range(_RPS)]
    for d_ in descs:
        d_.wait()
    plsc.subcore_barrier()
    pltpu.sync_copy(deg_s.at[pl.ds(s * RPT, RPT)],
                    out.at[c, pl.ds(s * RPT, RPT)])


@functools.partial(
    pl.kernel,
    out_type=jax.ShapeDtypeStruct((NC, NP, D), jnp.float32),
    mesh=_mesh,
    scratch_types=[
        pltpu.VMEM((4, NSUB, SUB), jnp.int32),
        pltpu.VMEM((4, NSUB, SUB), jnp.int32),
        pltpu.VMEM((4, CH), jnp.float32),
        pltpu.VMEM((3, CH, D), jnp.float32),
        pltpu.VMEM_SHARED((NP, D), jnp.float32),
        pltpu.SemaphoreType.DMA,
        pltpu.SemaphoreType.DMA,
        pltpu.SemaphoreType.DMA,
        pltpu.SemaphoreType.DMA,
        pltpu.SemaphoreType.DMA,
        pltpu.SemaphoreType.DMA,
        pltpu.SemaphoreType.DMA,
        pltpu.SemaphoreType.DMA,
        pltpu.SemaphoreType.DMA,
        pltpu.SemaphoreType.DMA,
    ],
    compiler_params=pltpu.CompilerParams(use_tc_tiling_on_sc=False),
)
def _prop_kernel(h, row2, col2, ew1, out, ridx_v, cidx_v, ew_v, rows_v,
                 acc_s, g0, g1, g2, s0, s1, s2, l0, l1, l2, l3):
    gsem = (g0, g1, g2)
    ssem = (s0, s1, s2)
    lsem = (l0, l1, l2, l3)
    c = lax.axis_index("c")
    s = lax.axis_index("s")
    wid = s * NC + c

    def _z(i, carry):
        rows_v[0, i, :] = jnp.zeros((D,), jnp.float32)
        return carry

    lax.fori_loop(0, RPT, _z, 0)
    pltpu.sync_copy(rows_v.at[0, pl.ds(0, RPT)], acc_s.at[pl.ds(s * RPT, RPT)])

    ldescs, gdescs, sdescs = {}, {}, {}

    def _issue_l(k):
        b = k % 4
        rb = wid * _RPS + k * NSUB
        ldescs[k] = [
            pltpu.async_copy(row2.at[pl.ds(rb, NSUB)], ridx_v.at[b], lsem[b]),
            pltpu.async_copy(col2.at[pl.ds(rb, NSUB)], cidx_v.at[b], lsem[b]),
            pltpu.async_copy(ew1.at[pl.ds(wid * EPW + k * CH, CH)],
                             ew_v.at[b], lsem[b]),
        ]

    def _issue_g(k):
        b4, b3 = k % 4, k % 3
        gdescs[k] = [
            pltpu.async_copy(h.at[ridx_v.at[b4, j]],
                             rows_v.at[b3, pl.ds(j * SUB, SUB)], gsem[b3])
            for j in range(NSUB)
        ]

    def _issue_s(k):
        b4, b3 = k % 4, k % 3
        sdescs[k] = [
            pltpu.async_copy(rows_v.at[b3, pl.ds(j * SUB, SUB)],
                             acc_s.at[cidx_v.at[b4, j]], ssem[b3], add=True)
            for j in range(NSUB)
        ]

    def _drain(descs):
        for d_ in descs:
            d_.wait()

    _issue_l(0)
    _issue_l(1)
    _drain(ldescs[0])
    _issue_g(0)
    plsc.subcore_barrier()

    for k in range(NCH):
        b4, b3 = k % 4, k % 3
        if k >= 2:
            _drain(sdescs[k - 2])
        if k + 1 < NCH:
            _drain(ldescs[k + 1])
            _issue_g(k + 1)
        if k + 2 < NCH:
            _issue_l(k + 2)
        _drain(gdescs[k])

        def _m(g, carry2, b4=b4, b3=b3):
            ew16 = ew_v[b4, pl.ds(g * 16, 16)]
            base = g * 16
            for e in range(16):
                rows_v[b3, base + e, :] = rows_v[b3, base + e, :] * ew16[e]
            return carry2

        lax.fori_loop(0, CH // 16, _m, 0)
        _issue_s(k)

    _drain(sdescs[NCH - 2])
    _drain(sdescs[NCH - 1])
    plsc.subcore_barrier()
    pltpu.sync_copy(acc_s.at[pl.ds(s * RPT, RPT)],
                    out.at[c, pl.ds(s * RPT, RPT)])


def _tc1_body(x_ref, w1_ref, dega_ref, degb_ref, h1p_ref, dis16_ref):
    deg = dega_ref[...] + degb_ref[...] + 1.0
    dis = lax.rsqrt(deg)
    h = jnp.dot(x_ref[...], w1_ref[...], preferred_element_type=jnp.float32)
    h1p_ref[...] = dis * h
    dis16_ref[...] = jnp.broadcast_to(dis, (N, D))


_tc1 = pl.pallas_call(
    _tc1_body,
    out_shape=(jax.ShapeDtypeStruct((N, D), jnp.float32),
               jax.ShapeDtypeStruct((N, D), jnp.float32)),
)


def _tc2_body(p1a_ref, p1b_ref, h1p_ref, dis16_ref, w2_ref, b1_ref, h2p_ref):
    s = p1a_ref[...] + p1b_ref[...] + h1p_ref[...]
    out1 = dis16_ref[...] * s + b1_ref[...]
    a = jnp.maximum(out1, 0.0)
    h2p_ref[...] = dis16_ref[...] * jnp.dot(
        a, w2_ref[...], preferred_element_type=jnp.float32)


_tc2 = pl.pallas_call(
    _tc2_body,
    out_shape=jax.ShapeDtypeStruct((N, D), jnp.float32),
)


def _tc3_body(p2a_ref, p2b_ref, h2p_ref, dis16_ref, b2_ref, out_ref):
    z = dis16_ref[...] * (p2a_ref[...] + p2b_ref[...] + h2p_ref[...])
    z = z + b2_ref[...]
    m = jnp.max(z, axis=1, keepdims=True)
    lse = jnp.log(jnp.sum(jnp.exp(z - m), axis=1, keepdims=True)) + m
    out_ref[...] = z - lse


_tc3 = pl.pallas_call(
    _tc3_body,
    out_shape=jax.ShapeDtypeStruct((N, D), jnp.float32),
)


def kernel(x, edge_index, edge_weight, W1, b1, W2, b2):
    row2 = edge_index[0].astype(jnp.int32).reshape(E // SUB, SUB)
    col2 = edge_index[1].astype(jnp.int32).reshape(E // SUB, SUB)
    ew2 = edge_weight.reshape(E // SUB, SUB)

    degp = _deg_kernel(col2, ew2)
    dega = degp[0, :N].reshape(N, 1)
    degb = degp[1, :N].reshape(N, 1)
    h1p, dis16 = _tc1(x, W1, dega, degb)

    ew1 = edge_weight.reshape(E)
    p1 = _prop_kernel(h1p, row2, col2, ew1)
    h2p = _tc2(p1[0, :N], p1[1, :N], h1p, dis16, W2, b1.reshape(1, D))

    p2 = _prop_kernel(h2p, row2, col2, ew1)
    out = _tc3(p2[0, :N], p2[1, :N], h2p, dis16, b2.reshape(1, D))
    return out


# speedup vs baseline: 55.8877x; 1.0417x over previous
import functools

import jax
import jax.numpy as jnp
from jax import lax
from jax.experimental import pallas as pl
from jax.experimental.pallas import tpu as pltpu
from jax.experimental.pallas import tpu_sc as plsc

N = 10000
E = 320000
DIN = 128
D = 16
NC = 2
NS = 16
NW = NC * NS
CH = 2000
EPW = E // NW
NCH = EPW // CH
RPT = 640
NP = NS * RPT
BLK = 2000

_mesh = plsc.VectorSubcoreMesh(
    core_axis_name="c", subcore_axis_name="s", num_cores=NC, num_subcores=NS)


@functools.partial(
    pl.kernel,
    out_type=jax.ShapeDtypeStruct((NC, NP), jnp.float32),
    mesh=_mesh,
    scratch_types=[
        pltpu.VMEM((EPW,), jnp.int32),
        pltpu.VMEM((EPW,), jnp.float32),
        pltpu.VMEM((RPT,), jnp.float32),
        pltpu.VMEM_SHARED((NP,), jnp.float32),
        pltpu.SemaphoreType.DMA,
    ],
)
def _deg_kernel(col1, ew1, out, cidx_v, ew_v, zbuf, deg_s, sem):
    c = lax.axis_index("c")
    s = lax.axis_index("s")
    wid = s * NC + c

    def _z(i, carry):
        zbuf[pl.ds(i * 16, 16)] = jnp.zeros((16,), jnp.float32)
        return carry

    lax.fori_loop(0, RPT // 16, _z, 0)
    pltpu.sync_copy(zbuf, deg_s.at[pl.ds(s * RPT, RPT)])
    pltpu.sync_copy(col1.at[pl.ds(wid * EPW, EPW)], cidx_v)
    pltpu.sync_copy(ew1.at[pl.ds(wid * EPW, EPW)], ew_v)
    plsc.subcore_barrier()
    pltpu.async_copy(ew_v, deg_s.at[cidx_v], sem, add=True).wait()
    plsc.subcore_barrier()
    pltpu.sync_copy(deg_s.at[pl.ds(s * RPT, RPT)],
                    out.at[c, pl.ds(s * RPT, RPT)])


@functools.partial(
    pl.kernel,
    out_type=jax.ShapeDtypeStruct((NC, NP, D), jnp.float32),
    mesh=_mesh,
    scratch_types=[
        pltpu.VMEM((4, CH), jnp.int32),
        pltpu.VMEM((4, CH), jnp.int32),
        pltpu.VMEM((4, CH), jnp.float32),
        pltpu.VMEM((3, CH, D), jnp.float32),
        pltpu.VMEM_SHARED((NP, D), jnp.float32),
        pltpu.SemaphoreType.DMA,
        pltpu.SemaphoreType.DMA,
        pltpu.SemaphoreType.DMA,
        pltpu.SemaphoreType.DMA,
        pltpu.SemaphoreType.DMA,
        pltpu.SemaphoreType.DMA,
        pltpu.SemaphoreType.DMA,
        pltpu.SemaphoreType.DMA,
        pltpu.SemaphoreType.DMA,
        pltpu.SemaphoreType.DMA,
    ],
    compiler_params=pltpu.CompilerParams(use_tc_tiling_on_sc=False),
)
def _prop_kernel(h, row1, col1, ew1, out, ridx_v, cidx_v, ew_v, rows_v,
                 acc_s, g0, g1, g2, s0, s1, s2, l0, l1, l2, l3):
    gsem = (g0, g1, g2)
    ssem = (s0, s1, s2)
    lsem = (l0, l1, l2, l3)
    c = lax.axis_index("c")
    s = lax.axis_index("s")
    wid = s * NC + c

    def _z(i, carry):
        rows_v[0, i, :] = jnp.zeros((D,), jnp.float32)
        return carry

    lax.fori_loop(0, RPT, _z, 0)
    pltpu.sync_copy(rows_v.at[0, pl.ds(0, RPT)], acc_s.at[pl.ds(s * RPT, RPT)])

    ldescs, gdescs, sdescs = {}, {}, {}

    def _issue_l(k):
        b = k % 4
        eb = wid * EPW + k * CH
        ldescs[k] = [
            pltpu.async_copy(row1.at[pl.ds(eb, CH)], ridx_v.at[b], lsem[b]),
            pltpu.async_copy(col1.at[pl.ds(eb, CH)], cidx_v.at[b], lsem[b]),
            pltpu.async_copy(ew1.at[pl.ds(eb, CH)], ew_v.at[b], lsem[b]),
        ]

    def _issue_g(k):
        b4, b3 = k % 4, k % 3
        gdescs[k] = [
            pltpu.async_copy(h.at[ridx_v.at[b4]], rows_v.at[b3], gsem[b3])
        ]

    def _issue_s(k):
        b4, b3 = k % 4, k % 3
        sdescs[k] = [
            pltpu.async_copy(rows_v.at[b3], acc_s.at[cidx_v.at[b4]],
                             ssem[b3], add=True)
        ]

    def _drain(descs):
        for d_ in descs:
            d_.wait()

    _issue_l(0)
    _issue_l(1)
    _drain(ldescs[0])
    _issue_g(0)
    plsc.subcore_barrier()

    for k in range(NCH):
        b4, b3 = k % 4, k % 3
        if k >= 2:
            _drain(sdescs[k - 2])
        if k + 1 < NCH:
            _drain(ldescs[k + 1])
            _issue_g(k + 1)
        if k + 2 < NCH:
            _issue_l(k + 2)
        _drain(gdescs[k])

        def _m(g, carry2, b4=b4, b3=b3):
            ew16 = ew_v[b4, pl.ds(g * 16, 16)]
            base = g * 16
            for e in range(16):
                rows_v[b3, base + e, :] = rows_v[b3, base + e, :] * ew16[e]
            return carry2

        lax.fori_loop(0, CH // 16, _m, 0)
        _issue_s(k)

    _drain(sdescs[NCH - 2])
    _drain(sdescs[NCH - 1])
    plsc.subcore_barrier()
    pltpu.sync_copy(acc_s.at[pl.ds(s * RPT, RPT)],
                    out.at[c, pl.ds(s * RPT, RPT)])


def _tc1_body(x_ref, w1_ref, dega_ref, degb_ref, h1p_ref):
    dis = lax.rsqrt(dega_ref[...] + degb_ref[...] + 1.0)
    h1p_ref[...] = dis * jnp.dot(x_ref[...], w1_ref[...],
                                 preferred_element_type=jnp.float32)


_tc1 = pl.pallas_call(
    _tc1_body,
    grid=(N // BLK,),
    in_specs=[
        pl.BlockSpec((BLK, DIN), lambda i: (i, 0)),
        pl.BlockSpec((DIN, D), lambda i: (0, 0)),
        pl.BlockSpec((BLK, 1), lambda i: (i, 0)),
        pl.BlockSpec((BLK, 1), lambda i: (i, 0)),
    ],
    out_specs=pl.BlockSpec((BLK, D), lambda i: (i, 0)),
    out_shape=jax.ShapeDtypeStruct((N, D), jnp.float32),
)


def _tc2_body(p1a_ref, p1b_ref, h1p_ref, dega_ref, degb_ref, w2_ref, b1_ref,
              h2p_ref):
    dis = lax.rsqrt(dega_ref[...] + degb_ref[...] + 1.0)
    out1 = dis * (p1a_ref[...] + p1b_ref[...] + h1p_ref[...]) + b1_ref[...]
    a = jnp.maximum(out1, 0.0)
    h2p_ref[...] = dis * jnp.dot(a, w2_ref[...],
                                 preferred_element_type=jnp.float32)


_tc2 = pl.pallas_call(
    _tc2_body,
    grid=(N // BLK,),
    in_specs=[
        pl.BlockSpec((BLK, D), lambda i: (i, 0)),
        pl.BlockSpec((BLK, D), lambda i: (i, 0)),
        pl.BlockSpec((BLK, D), lambda i: (i, 0)),
        pl.BlockSpec((BLK, 1), lambda i: (i, 0)),
        pl.BlockSpec((BLK, 1), lambda i: (i, 0)),
        pl.BlockSpec((D, D), lambda i: (0, 0)),
        pl.BlockSpec((1, D), lambda i: (0, 0)),
    ],
    out_specs=pl.BlockSpec((BLK, D), lambda i: (i, 0)),
    out_shape=jax.ShapeDtypeStruct((N, D), jnp.float32),
)


def _tc3_body(p2a_ref, p2b_ref, h2p_ref, dega_ref, degb_ref, b2_ref, out_ref):
    dis = lax.rsqrt(dega_ref[...] + degb_ref[...] + 1.0)
    z = dis * (p2a_ref[...] + p2b_ref[...] + h2p_ref[...]) + b2_ref[...]
    m = jnp.max(z, axis=1, keepdims=True)
    lse = jnp.log(jnp.sum(jnp.exp(z - m), axis=1, keepdims=True)) + m
    out_ref[...] = z - lse


_tc3 = pl.pallas_call(
    _tc3_body,
    grid=(N // BLK,),
    in_specs=[
        pl.BlockSpec((BLK, D), lambda i: (i, 0)),
        pl.BlockSpec((BLK, D), lambda i: (i, 0)),
        pl.BlockSpec((BLK, D), lambda i: (i, 0)),
        pl.BlockSpec((BLK, 1), lambda i: (i, 0)),
        pl.BlockSpec((BLK, 1), lambda i: (i, 0)),
        pl.BlockSpec((1, D), lambda i: (0, 0)),
    ],
    out_specs=pl.BlockSpec((BLK, D), lambda i: (i, 0)),
    out_shape=jax.ShapeDtypeStruct((N, D), jnp.float32),
)


def kernel(x, edge_index, edge_weight, W1, b1, W2, b2):
    row1 = edge_index[0].astype(jnp.int32)
    col1 = edge_index[1].astype(jnp.int32)

    degp = _deg_kernel(col1, edge_weight)
    dega = degp[0, :N].reshape(N, 1)
    degb = degp[1, :N].reshape(N, 1)
    h1p = _tc1(x, W1, dega, degb)

    p1 = _prop_kernel(h1p, row1, col1, edge_weight)
    h2p = _tc2(p1[0, :N], p1[1, :N], h1p, dega, degb, W2, b1.reshape(1, D))

    p2 = _prop_kernel(h2p, row1, col1, edge_weight)
    out = _tc3(p2[0, :N], p2[1, :N], h2p, dega, degb, b2.reshape(1, D))
    return out


# speedup vs baseline: 66.6740x; 1.1930x over previous
import functools

import jax
import jax.numpy as jnp
from jax import lax
from jax.experimental import pallas as pl
from jax.experimental.pallas import tpu as pltpu
from jax.experimental.pallas import tpu_sc as plsc

N = 10000
E = 320000
DIN = 128
D = 16
NC = 2
NS = 16
NW = NC * NS
CH = 2000
EPW = E // NW
NCH = EPW // CH
RPT = 640
NP = NS * RPT
BLK = 1000

_mesh = plsc.VectorSubcoreMesh(
    core_axis_name="c", subcore_axis_name="s", num_cores=NC, num_subcores=NS)


@functools.partial(
    pl.kernel,
    out_type=jax.ShapeDtypeStruct((NC, NP), jnp.float32),
    mesh=_mesh,
    scratch_types=[
        pltpu.VMEM((EPW,), jnp.int32),
        pltpu.VMEM((EPW,), jnp.float32),
        pltpu.VMEM((RPT,), jnp.float32),
        pltpu.VMEM_SHARED((NP,), jnp.float32),
        pltpu.SemaphoreType.DMA,
    ],
)
def _deg_kernel(col1, ew1, out, cidx_v, ew_v, zbuf, deg_s, sem):
    c = lax.axis_index("c")
    s = lax.axis_index("s")
    wid = s * NC + c

    def _z(i, carry):
        zbuf[pl.ds(i * 16, 16)] = jnp.zeros((16,), jnp.float32)
        return carry

    lax.fori_loop(0, RPT // 16, _z, 0)
    pltpu.sync_copy(zbuf, deg_s.at[pl.ds(s * RPT, RPT)])
    pltpu.sync_copy(col1.at[pl.ds(wid * EPW, EPW)], cidx_v)
    pltpu.sync_copy(ew1.at[pl.ds(wid * EPW, EPW)], ew_v)
    plsc.subcore_barrier()
    pltpu.async_copy(ew_v, deg_s.at[cidx_v], sem, add=True).wait()
    plsc.subcore_barrier()
    pltpu.sync_copy(deg_s.at[pl.ds(s * RPT, RPT)],
                    out.at[c, pl.ds(s * RPT, RPT)])


@functools.partial(
    pl.kernel,
    out_type=jax.ShapeDtypeStruct((NC, NP, D), jnp.float32),
    mesh=_mesh,
    scratch_types=[
        pltpu.VMEM((4, CH), jnp.int32),
        pltpu.VMEM((4, CH), jnp.int32),
        pltpu.VMEM((4, CH), jnp.float32),
        pltpu.VMEM((3, CH, D), jnp.float32),
        pltpu.VMEM_SHARED((NP, D), jnp.float32),
        pltpu.SemaphoreType.DMA,
        pltpu.SemaphoreType.DMA,
        pltpu.SemaphoreType.DMA,
        pltpu.SemaphoreType.DMA,
        pltpu.SemaphoreType.DMA,
        pltpu.SemaphoreType.DMA,
        pltpu.SemaphoreType.DMA,
        pltpu.SemaphoreType.DMA,
        pltpu.SemaphoreType.DMA,
        pltpu.SemaphoreType.DMA,
    ],
    compiler_params=pltpu.CompilerParams(use_tc_tiling_on_sc=False),
)
def _prop_kernel(h, row1, col1, ew1, out, ridx_v, cidx_v, ew_v, rows_v,
                 acc_s, g0, g1, g2, s0, s1, s2, l0, l1, l2, l3):
    gsem = (g0, g1, g2)
    ssem = (s0, s1, s2)
    lsem = (l0, l1, l2, l3)
    c = lax.axis_index("c")
    s = lax.axis_index("s")
    wid = s * NC + c

    def _z(i, carry):
        rows_v[0, i, :] = jnp.zeros((D,), jnp.float32)
        return carry

    lax.fori_loop(0, RPT, _z, 0)
    pltpu.sync_copy(rows_v.at[0, pl.ds(0, RPT)], acc_s.at[pl.ds(s * RPT, RPT)])

    ldescs, gdescs, sdescs = {}, {}, {}

    def _issue_l(k):
        b = k % 4
        eb = wid * EPW + k * CH
        ldescs[k] = [
            pltpu.async_copy(row1.at[pl.ds(eb, CH)], ridx_v.at[b], lsem[b]),
            pltpu.async_copy(col1.at[pl.ds(eb, CH)], cidx_v.at[b], lsem[b]),
            pltpu.async_copy(ew1.at[pl.ds(eb, CH)], ew_v.at[b], lsem[b]),
        ]

    def _issue_g(k):
        b4, b3 = k % 4, k % 3
        gdescs[k] = [
            pltpu.async_copy(h.at[ridx_v.at[b4]], rows_v.at[b3], gsem[b3])
        ]

    def _issue_s(k):
        b4, b3 = k % 4, k % 3
        sdescs[k] = [
            pltpu.async_copy(rows_v.at[b3], acc_s.at[cidx_v.at[b4]],
                             ssem[b3], add=True)
        ]

    def _drain(descs):
        for d_ in descs:
            d_.wait()

    _issue_l(0)
    _issue_l(1)
    _drain(ldescs[0])
    _issue_g(0)
    plsc.subcore_barrier()

    for k in range(NCH):
        b4, b3 = k % 4, k % 3
        if k >= 2:
            _drain(sdescs[k - 2])
        if k + 1 < NCH:
            _drain(ldescs[k + 1])
            _issue_g(k + 1)
        if k + 2 < NCH:
            _issue_l(k + 2)
        _drain(gdescs[k])

        def _m(g, carry2, b4=b4, b3=b3):
            ew16 = ew_v[b4, pl.ds(g * 16, 16)]
            base = g * 16
            for e in range(16):
                rows_v[b3, base + e, :] = rows_v[b3, base + e, :] * ew16[e]
            return carry2

        lax.fori_loop(0, CH // 16, _m, 0)
        _issue_s(k)

    _drain(sdescs[NCH - 2])
    _drain(sdescs[NCH - 1])
    plsc.subcore_barrier()
    pltpu.sync_copy(acc_s.at[pl.ds(s * RPT, RPT)],
                    out.at[c, pl.ds(s * RPT, RPT)])


ESPLIT = E // 10


def _split_body(ei_ref, row_ref, col_ref):
    row_ref[...] = ei_ref[0, :]
    col_ref[...] = ei_ref[1, :]


_split = pl.pallas_call(
    _split_body,
    out_shape=(jax.ShapeDtypeStruct((E,), jnp.int32),
               jax.ShapeDtypeStruct((E,), jnp.int32)),
)


def _tcdis_body(degp_ref, dis16_ref):
    dis = lax.rsqrt(degp_ref[0, :] + degp_ref[1, :] + 1.0)
    dis16_ref[...] = jnp.broadcast_to(dis[:, None], (NP, D))


_tcdis = pl.pallas_call(
    _tcdis_body,
    out_shape=jax.ShapeDtypeStruct((NP, D), jnp.float32),
)


def _tc1_body(x_ref, w1_ref, dis16_ref, h1p_ref):
    h1p_ref[...] = dis16_ref[...] * jnp.dot(
        x_ref[...], w1_ref[...], preferred_element_type=jnp.float32)


_tc1 = pl.pallas_call(
    _tc1_body,
    grid=(N // BLK,),
    in_specs=[
        pl.BlockSpec((BLK, DIN), lambda i: (i, 0)),
        pl.BlockSpec((DIN, D), lambda i: (0, 0)),
        pl.BlockSpec((BLK, D), lambda i: (i, 0)),
    ],
    out_specs=pl.BlockSpec((BLK, D), lambda i: (i, 0)),
    out_shape=jax.ShapeDtypeStruct((N, D), jnp.float32),
)


def _tc2_body(p1_ref, h1p_ref, dis16_ref, w2_ref, b1_ref, h2p_ref):
    dis = dis16_ref[...]
    out1 = dis * (p1_ref[0] + p1_ref[1] + h1p_ref[...]) + b1_ref[...]
    a = jnp.maximum(out1, 0.0)
    h2p_ref[...] = dis * jnp.dot(a, w2_ref[...],
                                 preferred_element_type=jnp.float32)


_tc2 = pl.pallas_call(
    _tc2_body,
    grid=(N // BLK,),
    in_specs=[
        pl.BlockSpec((NC, BLK, D), lambda i: (0, i, 0)),
        pl.BlockSpec((BLK, D), lambda i: (i, 0)),
        pl.BlockSpec((BLK, D), lambda i: (i, 0)),
        pl.BlockSpec((D, D), lambda i: (0, 0)),
        pl.BlockSpec((1, D), lambda i: (0, 0)),
    ],
    out_specs=pl.BlockSpec((BLK, D), lambda i: (i, 0)),
    out_shape=jax.ShapeDtypeStruct((N, D), jnp.float32),
)


def _tc3_body(p2_ref, h2p_ref, dis16_ref, b2_ref, out_ref):
    z = dis16_ref[...] * (p2_ref[0] + p2_ref[1] + h2p_ref[...]) + b2_ref[...]
    m = jnp.max(z, axis=1, keepdims=True)
    lse = jnp.log(jnp.sum(jnp.exp(z - m), axis=1, keepdims=True)) + m
    out_ref[...] = z - lse


_tc3 = pl.pallas_call(
    _tc3_body,
    grid=(N // BLK,),
    in_specs=[
        pl.BlockSpec((NC, BLK, D), lambda i: (0, i, 0)),
        pl.BlockSpec((BLK, D), lambda i: (i, 0)),
        pl.BlockSpec((BLK, D), lambda i: (i, 0)),
        pl.BlockSpec((1, D), lambda i: (0, 0)),
    ],
    out_specs=pl.BlockSpec((BLK, D), lambda i: (i, 0)),
    out_shape=jax.ShapeDtypeStruct((N, D), jnp.float32),
)


def kernel(x, edge_index, edge_weight, W1, b1, W2, b2):
    row1, col1 = _split(edge_index.astype(jnp.int32))

    degp = _deg_kernel(col1, edge_weight)
    dis16 = _tcdis(degp)
    h1p = _tc1(x, W1, dis16)

    p1 = _prop_kernel(h1p, row1, col1, edge_weight)
    h2p = _tc2(p1, h1p, dis16, W2, b1.reshape(1, D))

    p2 = _prop_kernel(h2p, row1, col1, edge_weight)
    out = _tc3(p2, h2p, dis16, b2.reshape(1, D))
    return out


# speedup vs baseline: 67.4971x; 1.0123x over previous
import functools

import jax
import jax.numpy as jnp
from jax import lax
from jax.experimental import pallas as pl
from jax.experimental.pallas import tpu as pltpu
from jax.experimental.pallas import tpu_sc as plsc

N = 10000
E = 320000
DIN = 128
D = 16
NC = 2
NS = 16
NW = NC * NS
CH = 2000
EPW = E // NW
NCH = EPW // CH
RPT = 640
NP = NS * RPT
BLK = 1024
PRB = BLK * D // 128
PR = NP * D // 128

_mesh = plsc.VectorSubcoreMesh(
    core_axis_name="c", subcore_axis_name="s", num_cores=NC, num_subcores=NS)


@functools.partial(
    pl.kernel,
    out_type=jax.ShapeDtypeStruct((NC, NP), jnp.float32),
    mesh=_mesh,
    scratch_types=[
        pltpu.VMEM((EPW,), jnp.int32),
        pltpu.VMEM((EPW,), jnp.float32),
        pltpu.VMEM((RPT,), jnp.float32),
        pltpu.VMEM_SHARED((NP,), jnp.float32),
        pltpu.SemaphoreType.DMA,
    ],
)
def _deg_kernel(col1, ew1, out, cidx_v, ew_v, zbuf, deg_s, sem):
    c = lax.axis_index("c")
    s = lax.axis_index("s")
    wid = s * NC + c

    def _z(i, carry):
        zbuf[pl.ds(i * 16, 16)] = jnp.zeros((16,), jnp.float32)
        return carry

    lax.fori_loop(0, RPT // 16, _z, 0)
    pltpu.sync_copy(zbuf, deg_s.at[pl.ds(s * RPT, RPT)])
    pltpu.sync_copy(col1.at[pl.ds(wid * EPW, EPW)], cidx_v)
    pltpu.sync_copy(ew1.at[pl.ds(wid * EPW, EPW)], ew_v)
    plsc.subcore_barrier()
    pltpu.async_copy(ew_v, deg_s.at[cidx_v], sem, add=True).wait()
    plsc.subcore_barrier()
    pltpu.sync_copy(deg_s.at[pl.ds(s * RPT, RPT)],
                    out.at[c, pl.ds(s * RPT, RPT)])


@functools.partial(
    pl.kernel,
    out_type=jax.ShapeDtypeStruct((NC, NP, D), jnp.float32),
    mesh=_mesh,
    scratch_types=[
        pltpu.VMEM((4, CH), jnp.int32),
        pltpu.VMEM((4, CH), jnp.int32),
        pltpu.VMEM((4, CH), jnp.float32),
        pltpu.VMEM((3, CH, D), jnp.float32),
        pltpu.VMEM_SHARED((NP, D), jnp.float32),
        pltpu.SemaphoreType.DMA,
        pltpu.SemaphoreType.DMA,
        pltpu.SemaphoreType.DMA,
        pltpu.SemaphoreType.DMA,
        pltpu.SemaphoreType.DMA,
        pltpu.SemaphoreType.DMA,
        pltpu.SemaphoreType.DMA,
        pltpu.SemaphoreType.DMA,
        pltpu.SemaphoreType.DMA,
        pltpu.SemaphoreType.DMA,
    ],
    compiler_params=pltpu.CompilerParams(use_tc_tiling_on_sc=False),
)
def _prop_kernel(h, row1, col1, ew1, out, ridx_v, cidx_v, ew_v, rows_v,
                 acc_s, g0, g1, g2, s0, s1, s2, l0, l1, l2, l3):
    gsem = (g0, g1, g2)
    ssem = (s0, s1, s2)
    lsem = (l0, l1, l2, l3)
    c = lax.axis_index("c")
    s = lax.axis_index("s")
    wid = s * NC + c

    def _z(i, carry):
        rows_v[0, i, :] = jnp.zeros((D,), jnp.float32)
        return carry

    lax.fori_loop(0, RPT, _z, 0)
    pltpu.sync_copy(rows_v.at[0, pl.ds(0, RPT)], acc_s.at[pl.ds(s * RPT, RPT)])

    ldescs, gdescs, sdescs = {}, {}, {}

    def _issue_l(k):
        b = k % 4
        eb = wid * EPW + k * CH
        ldescs[k] = [
            pltpu.async_copy(row1.at[pl.ds(eb, CH)], ridx_v.at[b], lsem[b]),
            pltpu.async_copy(col1.at[pl.ds(eb, CH)], cidx_v.at[b], lsem[b]),
            pltpu.async_copy(ew1.at[pl.ds(eb, CH)], ew_v.at[b], lsem[b]),
        ]

    def _issue_g(k):
        b4, b3 = k % 4, k % 3
        gdescs[k] = [
            pltpu.async_copy(h.at[ridx_v.at[b4]], rows_v.at[b3], gsem[b3])
        ]

    def _issue_s(k):
        b4, b3 = k % 4, k % 3
        sdescs[k] = [
            pltpu.async_copy(rows_v.at[b3], acc_s.at[cidx_v.at[b4]],
                             ssem[b3], add=True)
        ]

    def _drain(descs):
        for d_ in descs:
            d_.wait()

    _issue_l(0)
    _issue_l(1)
    _drain(ldescs[0])
    _issue_g(0)
    plsc.subcore_barrier()

    for k in range(NCH):
        b4, b3 = k % 4, k % 3
        if k >= 2:
            _drain(sdescs[k - 2])
        if k + 1 < NCH:
            _drain(ldescs[k + 1])
            _issue_g(k + 1)
        if k + 2 < NCH:
            _issue_l(k + 2)
        _drain(gdescs[k])

        def _m(g, carry2, b4=b4, b3=b3):
            ew16 = ew_v[b4, pl.ds(g * 16, 16)]
            base = g * 16
            for e in range(16):
                rows_v[b3, base + e, :] = rows_v[b3, base + e, :] * ew16[e]
            return carry2

        lax.fori_loop(0, CH // 16, _m, 0)
        _issue_s(k)

    _drain(sdescs[NCH - 2])
    _drain(sdescs[NCH - 1])
    plsc.subcore_barrier()
    pltpu.sync_copy(acc_s.at[pl.ds(s * RPT, RPT)],
                    out.at[c, pl.ds(s * RPT, RPT)])


ESPLIT = E // 10


def _split_body(ei_ref, row_ref, col_ref):
    row_ref[...] = ei_ref[0, :]
    col_ref[...] = ei_ref[1, :]


_split = pl.pallas_call(
    _split_body,
    out_shape=(jax.ShapeDtypeStruct((E,), jnp.int32),
               jax.ShapeDtypeStruct((E,), jnp.int32)),
)


def _tcdis_body(degp_ref, dis_ref):
    dis_ref[...] = lax.rsqrt(degp_ref[0, :] + degp_ref[1, :] + 1.0)


_tcdis = pl.pallas_call(
    _tcdis_body,
    out_shape=jax.ShapeDtypeStruct((NP,), jnp.float32),
)


def _tc1_body(x_ref, w1_ref, dis_ref, h1p_ref):
    h1p_ref[...] = dis_ref[...][:, None] * jnp.dot(
        x_ref[...], w1_ref[...], preferred_element_type=jnp.float32)


_tc1 = pl.pallas_call(
    _tc1_body,
    grid=(NP // BLK,),
    in_specs=[
        pl.BlockSpec((BLK, DIN), lambda i: (i, 0)),
        pl.BlockSpec((DIN, D), lambda i: (0, 0)),
        pl.BlockSpec((BLK,), lambda i: (i,)),
    ],
    out_specs=pl.BlockSpec((BLK, D), lambda i: (i, 0)),
    out_shape=jax.ShapeDtypeStruct((NP, D), jnp.float32),
)


def _tc2_body(p1_ref, h1p_ref, dis_ref, w2_ref, b1_ref, h2p_ref):
    sp = p1_ref[0] + p1_ref[1] + h1p_ref[...]
    dis = dis_ref[...][:, None]
    out1 = dis * sp + b1_ref[...]
    a = jnp.maximum(out1, 0.0)
    h2p_ref[...] = dis * jnp.dot(a, w2_ref[...],
                                 preferred_element_type=jnp.float32)


_tc2 = pl.pallas_call(
    _tc2_body,
    grid=(NP // BLK,),
    in_specs=[
        pl.BlockSpec((NC, BLK, D), lambda i: (0, i, 0)),
        pl.BlockSpec((BLK, D), lambda i: (i, 0)),
        pl.BlockSpec((BLK,), lambda i: (i,)),
        pl.BlockSpec((D, D), lambda i: (0, 0)),
        pl.BlockSpec((1, D), lambda i: (0, 0)),
    ],
    out_specs=pl.BlockSpec((BLK, D), lambda i: (i, 0)),
    out_shape=jax.ShapeDtypeStruct((NP, D), jnp.float32),
)


def _tc3_body(p2_ref, h2p_ref, dis_ref, b2_ref, out_ref):
    sp = p2_ref[0] + p2_ref[1] + h2p_ref[...]
    z = dis_ref[...][:, None] * sp + b2_ref[...]
    m = jnp.max(z, axis=1, keepdims=True)
    lse = jnp.log(jnp.sum(jnp.exp(z - m), axis=1, keepdims=True)) + m
    out_ref[...] = z - lse


_tc3 = pl.pallas_call(
    _tc3_body,
    grid=(NP // BLK,),
    in_specs=[
        pl.BlockSpec((NC, BLK, D), lambda i: (0, i, 0)),
        pl.BlockSpec((BLK, D), lambda i: (i, 0)),
        pl.BlockSpec((BLK,), lambda i: (i,)),
        pl.BlockSpec((1, D), lambda i: (0, 0)),
    ],
    out_specs=pl.BlockSpec((BLK, D), lambda i: (i, 0)),
    out_shape=jax.ShapeDtypeStruct((NP, D), jnp.float32),
)


def kernel(x, edge_index, edge_weight, W1, b1, W2, b2):
    row1, col1 = _split(edge_index.astype(jnp.int32))
    xp = jnp.pad(x, ((0, NP - N), (0, 0)))

    degp = _deg_kernel(col1, edge_weight)
    dis = _tcdis(degp)
    h1p = _tc1(xp, W1, dis)

    p1 = _prop_kernel(h1p, row1, col1, edge_weight)
    h2p = _tc2(p1, h1p, dis, W2, b1.reshape(1, D))

    p2 = _prop_kernel(h2p, row1, col1, edge_weight)
    out = _tc3(p2, h2p, dis, b2.reshape(1, D))
    return out[:N]


# speedup vs baseline: 73.7853x; 1.0932x over previous
import functools

import jax
import jax.numpy as jnp
from jax import lax
from jax.experimental import pallas as pl
from jax.experimental.pallas import tpu as pltpu
from jax.experimental.pallas import tpu_sc as plsc

N = 10000
E = 320000
DIN = 128
D = 16
NC = 2
NS = 16
NW = NC * NS
CH = 2000
EPW = E // NW
NCH = EPW // CH
RPT = 640
NP = NS * RPT
BLK = 1024
PRB = BLK * D // 128
PR = NP * D // 128

_mesh = plsc.VectorSubcoreMesh(
    core_axis_name="c", subcore_axis_name="s", num_cores=NC, num_subcores=NS)


@functools.partial(
    pl.kernel,
    out_type=jax.ShapeDtypeStruct((NC, NP), jnp.float32),
    mesh=_mesh,
    scratch_types=[
        pltpu.VMEM((EPW,), jnp.int32),
        pltpu.VMEM((EPW,), jnp.float32),
        pltpu.VMEM((RPT,), jnp.float32),
        pltpu.VMEM_SHARED((NP,), jnp.float32),
        pltpu.SemaphoreType.DMA,
    ],
)
def _deg_kernel(col1, ew1, out, cidx_v, ew_v, zbuf, deg_s, sem):
    c = lax.axis_index("c")
    s = lax.axis_index("s")
    wid = s * NC + c

    def _z(i, carry):
        zbuf[pl.ds(i * 16, 16)] = jnp.zeros((16,), jnp.float32)
        return carry

    lax.fori_loop(0, RPT // 16, _z, 0)
    pltpu.sync_copy(zbuf, deg_s.at[pl.ds(s * RPT, RPT)])
    pltpu.sync_copy(col1.at[pl.ds(wid * EPW, EPW)], cidx_v)
    pltpu.sync_copy(ew1.at[pl.ds(wid * EPW, EPW)], ew_v)
    plsc.subcore_barrier()
    pltpu.async_copy(ew_v, deg_s.at[cidx_v], sem, add=True).wait()
    plsc.subcore_barrier()
    pltpu.sync_copy(deg_s.at[pl.ds(s * RPT, RPT)],
                    out.at[c, pl.ds(s * RPT, RPT)])


_PROP_SCRATCH = [
    pltpu.VMEM((4, CH), jnp.int32),
    pltpu.VMEM((4, CH), jnp.int32),
    pltpu.VMEM((4, CH), jnp.float32),
    pltpu.VMEM((3, CH, D), jnp.float32),
    pltpu.VMEM_SHARED((NP, D), jnp.float32),
    pltpu.SemaphoreType.DMA,
    pltpu.SemaphoreType.DMA,
    pltpu.SemaphoreType.DMA,
    pltpu.SemaphoreType.DMA,
    pltpu.SemaphoreType.DMA,
    pltpu.SemaphoreType.DMA,
    pltpu.SemaphoreType.DMA,
    pltpu.SemaphoreType.DMA,
    pltpu.SemaphoreType.DMA,
    pltpu.SemaphoreType.DMA,
]


def _edge_pipeline(hsrc, row1, col1, ew1, out, ridx_v, cidx_v, ew_v, rows_v,
                   acc_s, gsem, ssem, lsem, c, s, wid):
    ldescs, gdescs, sdescs = {}, {}, {}

    def _issue_l(k):
        b = k % 4
        eb = wid * EPW + k * CH
        ldescs[k] = [
            pltpu.async_copy(row1.at[pl.ds(eb, CH)], ridx_v.at[b], lsem[b]),
            pltpu.async_copy(col1.at[pl.ds(eb, CH)], cidx_v.at[b], lsem[b]),
            pltpu.async_copy(ew1.at[pl.ds(eb, CH)], ew_v.at[b], lsem[b]),
        ]

    def _issue_g(k):
        b4, b3 = k % 4, k % 3
        gdescs[k] = [
            pltpu.async_copy(hsrc.at[ridx_v.at[b4]], rows_v.at[b3], gsem[b3])
        ]

    def _issue_s(k):
        b4, b3 = k % 4, k % 3
        sdescs[k] = [
            pltpu.async_copy(rows_v.at[b3], acc_s.at[cidx_v.at[b4]],
                             ssem[b3], add=True)
        ]

    def _drain(descs):
        for d_ in descs:
            d_.wait()

    _issue_l(0)
    _issue_l(1)
    _drain(ldescs[0])
    _issue_g(0)
    plsc.subcore_barrier()

    for k in range(NCH):
        b4, b3 = k % 4, k % 3
        if k >= 2:
            _drain(sdescs[k - 2])
        if k + 1 < NCH:
            _drain(ldescs[k + 1])
            _issue_g(k + 1)
        if k + 2 < NCH:
            _issue_l(k + 2)
        _drain(gdescs[k])

        def _m(g, carry2, b4=b4, b3=b3):
            ew16 = ew_v[b4, pl.ds(g * 16, 16)]
            base = g * 16
            for e in range(16):
                rows_v[b3, base + e, :] = rows_v[b3, base + e, :] * ew16[e]
            return carry2

        lax.fori_loop(0, CH // 16, _m, 0)
        _issue_s(k)

    _drain(sdescs[NCH - 2])
    _drain(sdescs[NCH - 1])
    plsc.subcore_barrier()
    pltpu.sync_copy(acc_s.at[pl.ds(s * RPT, RPT)],
                    out.at[c, pl.ds(s * RPT, RPT)])


@functools.partial(
    pl.kernel,
    out_type=jax.ShapeDtypeStruct((NC, NP, D), jnp.float32),
    mesh=_mesh,
    scratch_types=_PROP_SCRATCH,
    compiler_params=pltpu.CompilerParams(use_tc_tiling_on_sc=False),
)
def _prop1_kernel(h, row1, col1, ew1, out, ridx_v, cidx_v, ew_v, rows_v,
                  acc_s, g0, g1, g2, s0, s1, s2, l0, l1, l2, l3):
    c = lax.axis_index("c")
    s = lax.axis_index("s")
    wid = s * NC + c

    @pl.when(c == 0)
    def _seed():
        pltpu.sync_copy(h.at[pl.ds(s * RPT, RPT)],
                        acc_s.at[pl.ds(s * RPT, RPT)])

    @pl.when(c != 0)
    def _zero():
        def _z(i, carry):
            rows_v[0, i, :] = jnp.zeros((D,), jnp.float32)
            return carry

        lax.fori_loop(0, RPT, _z, 0)
        pltpu.sync_copy(rows_v.at[0, pl.ds(0, RPT)],
                        acc_s.at[pl.ds(s * RPT, RPT)])

    _edge_pipeline(h, row1, col1, ew1, out, ridx_v, cidx_v, ew_v, rows_v,
                   acc_s, (g0, g1, g2), (s0, s1, s2), (l0, l1, l2, l3),
                   c, s, wid)


@functools.partial(
    pl.kernel,
    out_type=jax.ShapeDtypeStruct((NP, D), jnp.float32),
    mesh=_mesh,
    scratch_types=[
        pltpu.VMEM((2 * RPT, D), jnp.float32),
        pltpu.VMEM((RPT, D), jnp.float32),
        pltpu.VMEM((RPT,), jnp.float32),
        pltpu.VMEM((D, D), jnp.float32),
        pltpu.VMEM((D,), jnp.float32),
    ],
    compiler_params=pltpu.CompilerParams(use_tc_tiling_on_sc=False),
)
def _mix_kernel(p1, dis, w2, b1, out, p_v, h2_v, dis_v, w2_v, b1_v):
    c = lax.axis_index("c")
    s = lax.axis_index("s")

    @pl.when(c == 0)
    def _go():
        base = s * RPT
        pltpu.sync_copy(p1.at[0, pl.ds(base, RPT)], p_v.at[pl.ds(0, RPT)])
        pltpu.sync_copy(p1.at[1, pl.ds(base, RPT)], p_v.at[pl.ds(RPT, RPT)])
        pltpu.sync_copy(dis.at[pl.ds(base, RPT)], dis_v)
        pltpu.sync_copy(w2, w2_v)
        pltpu.sync_copy(b1, b1_v)
        w2rows = [w2_v[k, :] for k in range(D)]
        b1vec = b1_v[...]

        def _row(g, carry):
            d16 = dis_v[pl.ds(g * 16, 16)]
            for e in range(16):
                r = g * 16 + e
                srow = p_v[r, :] + p_v[RPT + r, :]
                a = jnp.maximum(d16[e] * srow + b1vec, 0.0)
                acc = a[0] * w2rows[0]
                for k in range(1, D):
                    acc = acc + a[k] * w2rows[k]
                h2_v[r, :] = d16[e] * acc
            return carry

        lax.fori_loop(0, RPT // 16, _row, 0)
        pltpu.sync_copy(h2_v, out.at[pl.ds(base, RPT)])


ESPLIT = E // 10


def _split_body(ei_ref, row_ref, col_ref):
    row_ref[...] = ei_ref[0, :]
    col_ref[...] = ei_ref[1, :]


_split = pl.pallas_call(
    _split_body,
    out_shape=(jax.ShapeDtypeStruct((E,), jnp.int32),
               jax.ShapeDtypeStruct((E,), jnp.int32)),
)


def _tcdis_body(degp_ref, dis_ref):
    dis_ref[...] = lax.rsqrt(degp_ref[0, :] + degp_ref[1, :] + 1.0)


_tcdis = pl.pallas_call(
    _tcdis_body,
    out_shape=jax.ShapeDtypeStruct((NP,), jnp.float32),
)


def _tc1_body(x_ref, w1_ref, dis_ref, h1p_ref):
    h1p_ref[...] = dis_ref[...][:, None] * jnp.dot(
        x_ref[...], w1_ref[...], preferred_element_type=jnp.float32)


_tc1 = pl.pallas_call(
    _tc1_body,
    grid=(NP // BLK,),
    in_specs=[
        pl.BlockSpec((BLK, DIN), lambda i: (i, 0)),
        pl.BlockSpec((DIN, D), lambda i: (0, 0)),
        pl.BlockSpec((BLK,), lambda i: (i,)),
    ],
    out_specs=pl.BlockSpec((BLK, D), lambda i: (i, 0)),
    out_shape=jax.ShapeDtypeStruct((NP, D), jnp.float32),
)


def _tc3_body(p2_ref, dis_ref, b2_ref, out_ref):
    z = dis_ref[...][:, None] * (p2_ref[0] + p2_ref[1]) + b2_ref[...]
    m = jnp.max(z, axis=1, keepdims=True)
    lse = jnp.log(jnp.sum(jnp.exp(z - m), axis=1, keepdims=True)) + m
    out_ref[...] = z - lse


_tc3 = pl.pallas_call(
    _tc3_body,
    grid=(NP // BLK,),
    in_specs=[
        pl.BlockSpec((NC, BLK, D), lambda i: (0, i, 0)),
        pl.BlockSpec((BLK,), lambda i: (i,)),
        pl.BlockSpec((1, D), lambda i: (0, 0)),
    ],
    out_specs=pl.BlockSpec((BLK, D), lambda i: (i, 0)),
    out_shape=jax.ShapeDtypeStruct((NP, D), jnp.float32),
)


def kernel(x, edge_index, edge_weight, W1, b1, W2, b2):
    row1, col1 = _split(edge_index.astype(jnp.int32))
    xp = jnp.pad(x, ((0, NP - N), (0, 0)))

    degp = _deg_kernel(col1, edge_weight)
    dis = _tcdis(degp)
    h1p = _tc1(xp, W1, dis)

    p1 = _prop1_kernel(h1p, row1, col1, edge_weight)
    h2 = _mix_kernel(p1, dis, W2, b1)
    p2 = _prop1_kernel(h2, row1, col1, edge_weight)
    out = _tc3(p2, dis, b2.reshape(1, D))
    return out[:N]
